# baseline (speedup 1.0000x reference)
import functools

import jax
import jax.numpy as jnp
from jax.experimental import pallas as pl
from jax.experimental.pallas import tpu as pltpu
from jax.experimental.pallas import tpu_sc as plsc

_B = 16384
_D = 16
_WIN = 128
_BB = 2048


def _gather3(s_idx, l_idx, r_idx, s_tab, l_tab, r_tab):
    mesh = plsc.VectorSubcoreMesh(core_axis_name="core", subcore_axis_name="subcore")
    out_sds = jax.ShapeDtypeStruct((_B, _D), jnp.float32)

    @functools.partial(
        pl.kernel, out_type=(out_sds, out_sds, out_sds), mesh=mesh,
        compiler_params=pltpu.CompilerParams(use_tc_tiling_on_sc=False))
    def gather_kernel(si, li, ri, st, lt, rt, so, lo, ro):
        for idx_hbm, tab_hbm, out_hbm in ((si, st, so), (li, lt, lo), (ri, rt, ro)):
            def body(i_vmem, o_vmem, _tab=tab_hbm):
                pltpu.sync_copy(_tab.at[i_vmem.at[0]], o_vmem)

            pltpu.emit_pipeline(
                body,
                grid=(_B // _WIN,),
                in_specs=[pl.BlockSpec((1, _WIN), index_map=lambda i: (0, i))],
                out_specs=[pl.BlockSpec((_WIN, _D), index_map=lambda i: (i, 0))],
                core_axis_name=("core", "subcore"),
                dimension_semantics=(pltpu.PARALLEL,),
            )(idx_hbm, out_hbm)

    return gather_kernel(
        s_idx.reshape(1, _B), l_idx.reshape(1, _B), r_idx.reshape(1, _B),
        s_tab, l_tab, r_tab,
    )


def _mlp_body(s_ref, l_ref, r_ref, w1s_ref, w1l_ref, w1r_ref, b1_ref,
              w2_ref, b2_ref, w3_ref, b3_ref, o_ref):
    h = (jnp.dot(s_ref[...], w1s_ref[...], preferred_element_type=jnp.float32)
         + jnp.dot(l_ref[...], w1l_ref[...], preferred_element_type=jnp.float32)
         + jnp.dot(r_ref[...], w1r_ref[...], preferred_element_type=jnp.float32)
         + b1_ref[...])
    h = jnp.maximum(h, 0.0)
    h2 = jnp.dot(h, w2_ref[...], preferred_element_type=jnp.float32) + b2_ref[...]
    h2 = jnp.maximum(h2, 0.0)
    z = jnp.dot(h2, w3_ref[...], preferred_element_type=jnp.float32) + b3_ref[...]
    o_ref[...] = 1.0 / (1.0 + jnp.exp(-z))


def _mlp(s_rows, l_rows, r_rows, W1, b1, W2, b2, W3, b3):
    w1s, w1l, w1r = W1[0:16], W1[16:32], W1[32:48]
    b1r = b1.reshape(1, 64)
    b2r = b2.reshape(1, 32)
    b3r = b3.reshape(1, 1)
    full = lambda shape: pl.BlockSpec(shape, lambda i: (0, 0))
    return pl.pallas_call(
        _mlp_body,
        grid=(_B // _BB,),
        in_specs=[pl.BlockSpec((_BB, _D), lambda i: (i, 0))] * 3 + [
            full((16, 64)), full((16, 64)), full((16, 64)), full((1, 64)),
            full((64, 32)), full((1, 32)),
            full((32, 1)), full((1, 1)),
        ],
        out_specs=pl.BlockSpec((_BB, 1), lambda i: (i, 0)),
        out_shape=jax.ShapeDtypeStruct((_B, 1), jnp.float32),
    )(s_rows, l_rows, r_rows, w1s, w1l, w1r, b1r, W2, b2r, W3, b3r)


def kernel(skill_idx, location_idx, role_idx, skill_table, location_table,
           role_table, W1, b1, W2, b2, W3, b3):
    s_rows, l_rows, r_rows = _gather3(
        skill_idx.astype(jnp.int32), location_idx.astype(jnp.int32),
        role_idx.astype(jnp.int32), skill_table, location_table, role_table)
    out = _mlp(s_rows, l_rows, r_rows, W1, b1, W2, b2, W3, b3)
    return out[:, 0]
